# baseline (speedup 1.0000x reference)
import functools

import jax
import jax.numpy as jnp
from jax import lax
from jax.experimental import pallas as pl
from jax.experimental.pallas import tpu as pltpu
from jax.experimental.pallas import tpu_sc as plsc

B = 65536
NW = 32
BPW = B // NW
CH = 128
NCH = BPW // CH
L = 16
N = 100000


def _sc_coord_loss(pidx, btf, gtf):
    mesh = plsc.VectorSubcoreMesh(core_axis_name="c", subcore_axis_name="s")

    @functools.partial(
        pl.kernel,
        out_type=jax.ShapeDtypeStruct((NW, L), jnp.float32),
        mesh=mesh,
        compiler_params=pltpu.CompilerParams(
            needs_layout_passes=False, use_tc_tiling_on_sc=False),
        scratch_types=[
            pltpu.VMEM((BPW,), jnp.int32),
            pltpu.VMEM((BPW,), jnp.int32),
            pltpu.VMEM((4 * BPW,), jnp.float32),
            pltpu.VMEM((4 * BPW,), jnp.float32),
            pltpu.VMEM((L,), jnp.float32),
            pltpu.SemaphoreType.DMA,
            pltpu.SemaphoreType.DMA,
            pltpu.SemaphoreType.DMA,
            pltpu.SemaphoreType.DMA,
        ],
    )
    def body(idx_hbm, btf_hbm, gtf_hbm, out_hbm,
             pidx_v, gidx_v, pcol_v, gcol_v, acc_v,
             xysem_p, xysem_g, whsem_p, whsem_g):
        c = lax.axis_index("c")
        s = lax.axis_index("s")
        wid = s * 2 + c

        pltpu.sync_copy(idx_hbm.at[pl.ds(wid * BPW, BPW)], pidx_v)
        pltpu.sync_copy(idx_hbm.at[pl.ds(B + wid * BPW, BPW)], gidx_v)

        for k, (ps, gs) in enumerate([(xysem_p, xysem_g)] * 2
                                     + [(whsem_p, whsem_g)] * 2):
            src = btf_hbm.at[pl.ds(k * N, N)].at[pidx_v]
            pltpu.async_copy(src, pcol_v.at[pl.ds(k * BPW, BPW)], ps)
            src = gtf_hbm.at[pl.ds(k * N, N)].at[gidx_v]
            pltpu.async_copy(src, gcol_v.at[pl.ds(k * BPW, BPW)], gs)

        half = pl.ds(0, 2 * BPW)
        pltpu.make_async_copy(btf_hbm.at[half], pcol_v.at[half], xysem_p).wait()
        pltpu.make_async_copy(gtf_hbm.at[half], gcol_v.at[half], xysem_g).wait()

        def step_xy(j, acc):
            base = j * L
            px = pcol_v[pl.ds(base, L)]
            py = pcol_v[pl.ds(base + BPW, L)]
            gx = gcol_v[pl.ds(base, L)]
            gy = gcol_v[pl.ds(base + BPW, L)]
            return acc + jnp.abs(px - gx) + jnp.abs(py - gy)

        acc = lax.fori_loop(0, BPW // L, step_xy,
                            jnp.zeros((L,), jnp.float32), unroll=2)

        pltpu.make_async_copy(btf_hbm.at[half], pcol_v.at[half], whsem_p).wait()
        pltpu.make_async_copy(gtf_hbm.at[half], gcol_v.at[half], whsem_g).wait()

        def step_wh(j, acc):
            base = j * L
            px = pcol_v[pl.ds(base, L)]
            py = pcol_v[pl.ds(base + BPW, L)]
            pz = pcol_v[pl.ds(base + 2 * BPW, L)]
            pw = pcol_v[pl.ds(base + 3 * BPW, L)]
            gx = gcol_v[pl.ds(base, L)]
            gy = gcol_v[pl.ds(base + BPW, L)]
            gw = gcol_v[pl.ds(base + 2 * BPW, L)]
            gh = gcol_v[pl.ds(base + 3 * BPW, L)]
            t = jnp.abs(pz - (gx + gw)) + jnp.abs(pw - (gy + gh))
            return acc + t

        acc = lax.fori_loop(0, BPW // L, step_wh, acc, unroll=2)
        acc_v[...] = acc
        pltpu.sync_copy(acc_v, out_hbm.at[wid])

    return body(pidx, btf, gtf)


def kernel(boxes, gt, positive_idx):
    idx_flat = positive_idx.T.reshape(-1)
    partials = _sc_coord_loss(idx_flat, boxes.T.reshape(-1), gt.T.reshape(-1))
    return jnp.sum(partials) * (1.0 / (B * 4))


# speedup vs baseline: 1.0135x; 1.0135x over previous
import functools

import jax
import jax.numpy as jnp
from jax import lax
from jax.experimental import pallas as pl
from jax.experimental.pallas import tpu as pltpu
from jax.experimental.pallas import tpu_sc as plsc

B = 65536
NW = 32
BPW = B // NW
CH = 128
NCH = BPW // CH
L = 16
N = 100000


def _sc_coord_loss(pidx, btf, gtf):
    mesh = plsc.VectorSubcoreMesh(core_axis_name="c", subcore_axis_name="s")

    @functools.partial(
        pl.kernel,
        out_type=jax.ShapeDtypeStruct((NW, L), jnp.float32),
        mesh=mesh,
        compiler_params=pltpu.CompilerParams(
            needs_layout_passes=False, use_tc_tiling_on_sc=False),
        scratch_types=[
            pltpu.VMEM((BPW,), jnp.int32),
            pltpu.VMEM((BPW,), jnp.int32),
            pltpu.VMEM((4 * BPW,), jnp.float32),
            pltpu.VMEM((4 * BPW,), jnp.float32),
            pltpu.VMEM((L,), jnp.float32),
            pltpu.SemaphoreType.DMA,
            pltpu.SemaphoreType.DMA,
            pltpu.SemaphoreType.DMA,
        ],
    )
    def body(idx_hbm, btf_hbm, gtf_hbm, out_hbm,
             pidx_v, gidx_v, pcol_v, gcol_v, acc_v, psem, gsem, isem):
        c = lax.axis_index("c")
        s = lax.axis_index("s")
        wid = s * 2 + c

        stage_p = pltpu.async_copy(
            idx_hbm.at[pl.ds(wid * BPW, BPW)], pidx_v, isem)
        stage_g = pltpu.async_copy(
            idx_hbm.at[pl.ds(B + wid * BPW, BPW)], gidx_v, isem)

        stage_p.wait()
        for k in range(4):
            src = btf_hbm.at[pl.ds(k * N, N)].at[pidx_v]
            pltpu.async_copy(src, pcol_v.at[pl.ds(k * BPW, BPW)], psem)
        stage_g.wait()
        for k in range(4):
            src = gtf_hbm.at[pl.ds(k * N, N)].at[gidx_v]
            pltpu.async_copy(src, gcol_v.at[pl.ds(k * BPW, BPW)], gsem)

        pltpu.make_async_copy(btf_hbm.at[pl.ds(0, 4 * BPW)], pcol_v, psem).wait()
        pltpu.make_async_copy(gtf_hbm.at[pl.ds(0, 4 * BPW)], gcol_v, gsem).wait()

        def step(j, acc):
            base = j * L
            px = pcol_v[pl.ds(base, L)]
            py = pcol_v[pl.ds(base + BPW, L)]
            pz = pcol_v[pl.ds(base + 2 * BPW, L)]
            pw = pcol_v[pl.ds(base + 3 * BPW, L)]
            gx = gcol_v[pl.ds(base, L)]
            gy = gcol_v[pl.ds(base + BPW, L)]
            gw = gcol_v[pl.ds(base + 2 * BPW, L)]
            gh = gcol_v[pl.ds(base + 3 * BPW, L)]
            t = (jnp.abs(px - gx) + jnp.abs(py - gy)
                 + jnp.abs(pz - (gx + gw)) + jnp.abs(pw - (gy + gh)))
            return acc + t

        acc = lax.fori_loop(0, BPW // L, step, jnp.zeros((L,), jnp.float32))
        acc_v[...] = acc
        pltpu.sync_copy(acc_v, out_hbm.at[wid])

    return body(pidx, btf, gtf)


def kernel(boxes, gt, positive_idx):
    idx_flat = positive_idx.T.reshape(-1)
    partials = _sc_coord_loss(idx_flat, boxes.T.reshape(-1), gt.T.reshape(-1))
    return jnp.sum(partials) * (1.0 / (B * 4))
